# speedup vs baseline: 95.7305x; 95.7305x over previous
import jax
import jax.numpy as jnp
from jax import lax
from jax.experimental import pallas as pl
from jax.experimental.pallas import tpu as pltpu
from jax.experimental.pallas import tpu_sc as plsc

_B = 16384
_S = 200
_N = _B * _S
_NC = 2
_NS = 16
_NW = _NC * _NS
_PER_W = _N // _NW
_CHUNK = 12800
_NCHUNK = _PER_W // _CHUNK
_LANES = 16


def _gather_exp_body(x_hbm, table_hbm, out_hbm, idx_v, rows_v, sem):
    wid = lax.axis_index("s") * _NC + lax.axis_index("c")
    base = wid * _PER_W

    def chunk_body(ci, carry):
        off = base + ci * _CHUNK
        pltpu.sync_copy(x_hbm.at[pl.ds(off, _CHUNK)], idx_v)
        pltpu.async_copy(table_hbm.at[idx_v], rows_v, sem).wait()

        def exp_body(vi, c):
            sl = pl.ds(pl.multiple_of(vi * _LANES, _LANES), _LANES)
            rows_v[sl] = jnp.exp(rows_v[sl])
            return c

        lax.fori_loop(0, _CHUNK // _LANES, exp_body, 0)
        pltpu.sync_copy(rows_v, out_hbm.at[pl.ds(off, _CHUNK)])
        return carry

    lax.fori_loop(0, _NCHUNK, chunk_body, 0)


def kernel(x, table):
    xf = x.reshape(_N).astype(jnp.int32)
    tf = table.reshape(table.shape[0])
    mesh = plsc.VectorSubcoreMesh(core_axis_name="c", subcore_axis_name="s")
    fn = pl.kernel(
        _gather_exp_body,
        out_type=jax.ShapeDtypeStruct((_N,), jnp.float32),
        mesh=mesh,
        scratch_types=[
            pltpu.VMEM((_CHUNK,), jnp.int32),
            pltpu.VMEM((_CHUNK,), jnp.float32),
            pltpu.SemaphoreType.DMA,
        ],
    )
    out = fn(xf, tf)
    return out.reshape(_B, _S)


# speedup vs baseline: 138.5052x; 1.4468x over previous
import jax
import jax.numpy as jnp
from jax import lax
from jax.experimental import pallas as pl
from jax.experimental.pallas import tpu as pltpu
from jax.experimental.pallas import tpu_sc as plsc

_B = 16384
_S = 200
_N = _B * _S
_V = 1000000
_NC = 2
_NS = 16
_NW = _NC * _NS
_PER_W = _N // _NW
_CHUNK = 12800
_NCHUNK = _PER_W // _CHUNK
_LANES = 16
_STAGE_CHUNK = 10000
_N_STAGE_CHUNKS = _V // _STAGE_CHUNK
_STAGE_TRIPS = -(-_N_STAGE_CHUNKS // _NS)


def _gather_exp_body(x_hbm, table_hbm, out_hbm, stage_v, idx_v, rows_v,
                     etab_s, sem):
    c = lax.axis_index("c")
    s = lax.axis_index("s")
    wid = s * _NC + c

    def stage_body(k, carry):
        ci = k * _NS + s

        @pl.when(ci < _N_STAGE_CHUNKS)
        def _():
            off = ci * _STAGE_CHUNK
            pltpu.sync_copy(table_hbm.at[pl.ds(off, _STAGE_CHUNK)], stage_v)

            def exp_body(vi, cc):
                sl = pl.ds(pl.multiple_of(vi * _LANES, _LANES), _LANES)
                stage_v[sl] = jnp.exp(stage_v[sl])
                return cc

            lax.fori_loop(0, _STAGE_CHUNK // _LANES, exp_body, 0)
            pltpu.sync_copy(stage_v, etab_s.at[pl.ds(off, _STAGE_CHUNK)])

        return carry

    lax.fori_loop(0, _STAGE_TRIPS, stage_body, 0)
    plsc.subcore_barrier()

    base = wid * _PER_W

    def chunk_body(ci, carry):
        off = base + ci * _CHUNK
        pltpu.sync_copy(x_hbm.at[pl.ds(off, _CHUNK)], idx_v)
        pltpu.async_copy(etab_s.at[idx_v], rows_v, sem).wait()
        pltpu.sync_copy(rows_v, out_hbm.at[pl.ds(off, _CHUNK)])
        return carry

    lax.fori_loop(0, _NCHUNK, chunk_body, 0)


def kernel(x, table):
    xf = x.reshape(_N).astype(jnp.int32)
    tf = table.reshape(table.shape[0])
    mesh = plsc.VectorSubcoreMesh(core_axis_name="c", subcore_axis_name="s")
    fn = pl.kernel(
        _gather_exp_body,
        out_type=jax.ShapeDtypeStruct((_N,), jnp.float32),
        mesh=mesh,
        scratch_types=[
            pltpu.VMEM((_STAGE_CHUNK,), jnp.float32),
            pltpu.VMEM((_CHUNK,), jnp.int32),
            pltpu.VMEM((_CHUNK,), jnp.float32),
            pltpu.VMEM_SHARED((_V,), jnp.float32),
            pltpu.SemaphoreType.DMA,
        ],
    )
    out = fn(xf, tf)
    return out.reshape(_B, _S)


# speedup vs baseline: 138.7389x; 1.0017x over previous
import jax
import jax.numpy as jnp
from jax import lax
from jax.experimental import pallas as pl
from jax.experimental.pallas import tpu as pltpu
from jax.experimental.pallas import tpu_sc as plsc

_B = 16384
_S = 200
_N = _B * _S
_V = 1000000
_NC = 2
_NS = 16
_NW = _NC * _NS
_ELEMS_W = _N // _NW
_CHUNK = 12800
_NCHUNK = _ELEMS_W // _CHUNK
_LANES = 16
_STAGE_CHUNK = 10000
_N_STAGE_CHUNKS = _V // _STAGE_CHUNK
_STAGE_TRIPS = -(-_N_STAGE_CHUNKS // _NS)


def _gather_exp_body(x_hbm, table_hbm, out_hbm, stage_v, idx_v, rows_v,
                     etab_s, sem):
    c = lax.axis_index("c")
    s = lax.axis_index("s")
    wid = s * _NC + c

    def stage_body(k, carry):
        ci = k * _NS + s

        @pl.when(ci < _N_STAGE_CHUNKS)
        def _():
            off = ci * _STAGE_CHUNK
            pltpu.sync_copy(table_hbm.at[pl.ds(off, _STAGE_CHUNK)], stage_v)

            def exp_body(vi, cc):
                sl = pl.ds(pl.multiple_of(vi * _LANES, _LANES), _LANES)
                stage_v[sl] = jnp.exp(stage_v[sl])
                return cc

            lax.fori_loop(0, _STAGE_CHUNK // _LANES, exp_body, 0)
            pltpu.sync_copy(stage_v, etab_s.at[pl.ds(off, _STAGE_CHUNK)])

        return carry

    lax.fori_loop(0, _STAGE_TRIPS, stage_body, 0)
    plsc.subcore_barrier()

    base = wid * _ELEMS_W

    def chunk_body(ci, carry):
        e0 = base + ci * _CHUNK
        pltpu.sync_copy(x_hbm.at[pl.ds(e0, _CHUNK)], idx_v)
        pltpu.async_copy(etab_s.at[idx_v], rows_v, sem).wait()
        pltpu.sync_copy(rows_v, out_hbm.at[pl.ds(e0, _CHUNK)])
        return carry

    lax.fori_loop(0, _NCHUNK, chunk_body, 0)


def kernel(x, table):
    xi = x.astype(jnp.int32).reshape(_N)
    tf = table.reshape(table.shape[0])
    mesh = plsc.VectorSubcoreMesh(core_axis_name="c", subcore_axis_name="s")
    fn = pl.kernel(
        _gather_exp_body,
        out_type=jax.ShapeDtypeStruct((_N,), jnp.float32),
        mesh=mesh,
        scratch_types=[
            pltpu.VMEM((_STAGE_CHUNK,), jnp.float32),
            pltpu.VMEM((_CHUNK,), jnp.int32),
            pltpu.VMEM((_CHUNK,), jnp.float32),
            pltpu.VMEM_SHARED((_V,), jnp.float32),
            pltpu.SemaphoreType.DMA,
        ],
    )
    return fn(xi, tf).reshape(_B, _S)


# speedup vs baseline: 161.1144x; 1.1613x over previous
import jax
import jax.numpy as jnp
from jax import lax
from jax.experimental import pallas as pl
from jax.experimental.pallas import tpu as pltpu
from jax.experimental.pallas import tpu_sc as plsc

_B = 16384
_S = 200
_N = _B * _S
_V = 1000000
_NC = 2
_NS = 16
_NW = _NC * _NS
_ELEMS_W = _N // _NW
_CHUNK = 12800
_NCHUNK = _ELEMS_W // _CHUNK
_LANES = 16
_STAGE_CHUNK = 8192
_N_FULL_CHUNKS = _V // _STAGE_CHUNK
_STAGE_TAIL = _V - _N_FULL_CHUNKS * _STAGE_CHUNK
_N_STAGE_CHUNKS = _N_FULL_CHUNKS + 1
_STAGE_TRIPS = -(-_N_STAGE_CHUNKS // _NS)


def _gather_exp_body(x_hbm, table_hbm, out_hbm, stage_v, idx_v,
                     rows_v, etab_s, sem):
    c = lax.axis_index("c")
    s = lax.axis_index("s")
    wid = s * _NC + c

    def _stage(off, size):
        pltpu.sync_copy(table_hbm.at[0, pl.ds(off, size)],
                        stage_v.at[pl.ds(0, size)])

        def exp_body(vi, cc):
            sl = pl.ds(pl.multiple_of(vi * _LANES, _LANES), _LANES)
            stage_v[sl] = jnp.exp(stage_v[sl])
            return cc

        lax.fori_loop(0, size // _LANES, exp_body, 0)
        pltpu.sync_copy(stage_v.at[pl.ds(0, size)],
                        etab_s.at[pl.ds(off, size)])

    def stage_body(k, carry):
        ci = k * _NS + s

        @pl.when(ci < _N_FULL_CHUNKS)
        def _():
            _stage(ci * _STAGE_CHUNK, _STAGE_CHUNK)

        @pl.when(ci == _N_FULL_CHUNKS)
        def _():
            _stage(_N_FULL_CHUNKS * _STAGE_CHUNK, _STAGE_TAIL)

        return carry

    lax.fori_loop(0, _STAGE_TRIPS, stage_body, 0)
    plsc.subcore_barrier()

    base = wid * _ELEMS_W

    def chunk_body(ci, carry):
        e0 = base + ci * _CHUNK
        pltpu.sync_copy(x_hbm.at[pl.ds(e0, _CHUNK)], idx_v)
        pltpu.async_copy(etab_s.at[idx_v], rows_v, sem).wait()
        pltpu.sync_copy(rows_v, out_hbm.at[pl.ds(e0, _CHUNK)])
        return carry

    lax.fori_loop(0, _NCHUNK, chunk_body, 0)


def kernel(x, table):
    xi = x.reshape(_N)
    tt = table.T
    mesh = plsc.VectorSubcoreMesh(core_axis_name="c", subcore_axis_name="s")
    fn = pl.kernel(
        _gather_exp_body,
        out_type=jax.ShapeDtypeStruct((_N,), jnp.float32),
        mesh=mesh,
        scratch_types=[
            pltpu.VMEM((_STAGE_CHUNK,), jnp.float32),
            pltpu.VMEM((_CHUNK,), jnp.int32),
            pltpu.VMEM((_CHUNK,), jnp.float32),
            pltpu.VMEM_SHARED((_V,), jnp.float32),
            pltpu.SemaphoreType.DMA,
        ],
    )
    return fn(xi, tt).reshape(_B, _S)


# speedup vs baseline: 239.3233x; 1.4854x over previous
import jax
import jax.numpy as jnp
from jax import lax
from jax.experimental import pallas as pl
from jax.experimental.pallas import tpu as pltpu
from jax.experimental.pallas import tpu_sc as plsc

_B = 16384
_S = 200
_N = _B * _S
_V = 1000000
_NC = 2
_NS = 16
_NW = _NC * _NS
_SEG = 4096
_SEG_PER_ROW = _B // _SEG
_NSEG = _S * _SEG_PER_ROW
_SEG_W = _NSEG // _NW
_LANES = 16
_STAGE_CHUNK = 8192
_N_FULL_CHUNKS = _V // _STAGE_CHUNK
_STAGE_TAIL = _V - _N_FULL_CHUNKS * _STAGE_CHUNK
_N_STAGE_CHUNKS = _N_FULL_CHUNKS + 1
_STAGE_TRIPS = -(-_N_STAGE_CHUNKS // _NS)


def _gather_exp_body(x_hbm, table_hbm, out_hbm, stage_v, idx_v,
                     rows_v, etab_s, sem):
    c = lax.axis_index("c")
    s = lax.axis_index("s")
    wid = s * _NC + c

    def _stage(off, size):
        pltpu.sync_copy(table_hbm.at[0, pl.ds(off, size)],
                        stage_v.at[pl.ds(0, size)])

        def exp_body(vi, cc):
            sl = pl.ds(pl.multiple_of(vi * _LANES, _LANES), _LANES)
            stage_v[sl] = jnp.exp(stage_v[sl])
            return cc

        lax.fori_loop(0, size // _LANES, exp_body, 0)
        pltpu.sync_copy(stage_v.at[pl.ds(0, size)],
                        etab_s.at[pl.ds(off, size)])

    def stage_body(k, carry):
        ci = k * _NS + s

        @pl.when(ci < _N_FULL_CHUNKS)
        def _():
            _stage(ci * _STAGE_CHUNK, _STAGE_CHUNK)

        @pl.when(ci == _N_FULL_CHUNKS)
        def _():
            _stage(_N_FULL_CHUNKS * _STAGE_CHUNK, _STAGE_TAIL)

        return carry

    lax.fori_loop(0, _STAGE_TRIPS, stage_body, 0)
    plsc.subcore_barrier()

    base = wid * _SEG_W

    def seg_body(i, carry):
        q = base + i
        r = q // _SEG_PER_ROW
        b0 = (q % _SEG_PER_ROW) * _SEG
        pltpu.sync_copy(x_hbm.at[pl.ds(r, 1), pl.ds(b0, _SEG)], idx_v)
        pltpu.async_copy(etab_s.at[idx_v.at[0]], rows_v.at[0], sem).wait()
        pltpu.sync_copy(rows_v, out_hbm.at[pl.ds(r, 1), pl.ds(b0, _SEG)])
        return carry

    lax.fori_loop(0, _SEG_W, seg_body, 0)


def kernel(x, table):
    xt = x.T
    tt = table.T
    mesh = plsc.VectorSubcoreMesh(core_axis_name="c", subcore_axis_name="s")
    fn = pl.kernel(
        _gather_exp_body,
        out_type=jax.ShapeDtypeStruct((_S, _B), jnp.float32),
        mesh=mesh,
        scratch_types=[
            pltpu.VMEM((_STAGE_CHUNK,), jnp.float32),
            pltpu.VMEM((1, _SEG), jnp.int32),
            pltpu.VMEM((1, _SEG), jnp.float32),
            pltpu.VMEM_SHARED((_V,), jnp.float32),
            pltpu.SemaphoreType.DMA,
        ],
    )
    return fn(xt, tt).T


# speedup vs baseline: 281.3005x; 1.1754x over previous
import jax
import jax.numpy as jnp
from jax import lax
from jax.experimental import pallas as pl
from jax.experimental.pallas import tpu as pltpu
from jax.experimental.pallas import tpu_sc as plsc

_B = 16384
_S = 200
_N = _B * _S
_V = 1000000
_NC = 2
_NS = 16
_NW = _NC * _NS
_SEG = 2048
_SEG_PER_ROW = _B // _SEG
_NSEG = _S * _SEG_PER_ROW
_SEG_W = _NSEG // _NW
_PAIRS = _SEG_W // 2
_LANES = 16
_STAGE_CHUNK = 8192
_N_FULL_CHUNKS = _V // _STAGE_CHUNK
_STAGE_TAIL = _V - _N_FULL_CHUNKS * _STAGE_CHUNK
_N_STAGE_CHUNKS = _N_FULL_CHUNKS + 1
_STAGE_TRIPS = -(-_N_STAGE_CHUNKS // _NS)


def _gather_exp_body(x_hbm, table_hbm, out_hbm, stage_v, idx_a, idx_b,
                     rows_a, rows_b, etab_s, sem_g, sem_ia, sem_ib,
                     sem_oa, sem_ob):
    c = lax.axis_index("c")
    s = lax.axis_index("s")
    wid = s * _NC + c

    def _stage(off, size):
        pltpu.sync_copy(table_hbm.at[0, pl.ds(off, size)],
                        stage_v.at[pl.ds(0, size)])

        def exp_body(vi, cc):
            sl = pl.ds(pl.multiple_of(vi * _LANES, _LANES), _LANES)
            stage_v[sl] = jnp.exp(stage_v[sl])
            return cc

        lax.fori_loop(0, size // _LANES, exp_body, 0)
        pltpu.sync_copy(stage_v.at[pl.ds(0, size)],
                        etab_s.at[pl.ds(off, size)])

    def stage_body(k, carry):
        ci = k * _NS + s

        @pl.when(ci < _N_FULL_CHUNKS)
        def _():
            _stage(ci * _STAGE_CHUNK, _STAGE_CHUNK)

        @pl.when(ci == _N_FULL_CHUNKS)
        def _():
            _stage(_N_FULL_CHUNKS * _STAGE_CHUNK, _STAGE_TAIL)

        return carry

    lax.fori_loop(0, _STAGE_TRIPS, stage_body, 0)
    plsc.subcore_barrier()

    base = wid * _SEG_W

    def _x_slice(q):
        r = q // _SEG_PER_ROW
        b0 = (q % _SEG_PER_ROW) * _SEG
        return x_hbm.at[pl.ds(r, 1), pl.ds(b0, _SEG)]

    def _out_slice(q):
        r = q // _SEG_PER_ROW
        b0 = (q % _SEG_PER_ROW) * _SEG
        return out_hbm.at[pl.ds(r, 1), pl.ds(b0, _SEG)]

    def _run_seg(q, qnext, qprev, idx_c, rows_c, idx_n, rows_p, sem_ic,
                 sem_in, sem_oc, sem_op, j):
        pltpu.make_async_copy(_x_slice(q), idx_c, sem_ic).wait()

        @pl.when(qnext < base + _SEG_W)
        def _():
            pltpu.async_copy(_x_slice(qnext), idx_n, sem_in)

        @pl.when(j > 0)
        def _():
            pltpu.make_async_copy(rows_c, _out_slice(qprev), sem_oc).wait()

        pltpu.async_copy(etab_s.at[idx_c.at[0]], rows_c.at[0], sem_g).wait()
        pltpu.async_copy(rows_c, _out_slice(q), sem_oc)

    pltpu.async_copy(_x_slice(base), idx_a, sem_ia)

    def pair_body(j, carry):
        q0 = base + 2 * j
        _run_seg(q0, q0 + 1, q0 - 2, idx_a, rows_a, idx_b, rows_b,
                 sem_ia, sem_ib, sem_oa, sem_ob, j)
        _run_seg(q0 + 1, q0 + 2, q0 - 1, idx_b, rows_b, idx_a, rows_a,
                 sem_ib, sem_ia, sem_ob, sem_oa, j)
        return carry

    lax.fori_loop(0, _PAIRS, pair_body, 0)
    pltpu.make_async_copy(rows_a, _out_slice(base + _SEG_W - 2),
                          sem_oa).wait()
    pltpu.make_async_copy(rows_b, _out_slice(base + _SEG_W - 1),
                          sem_ob).wait()


def kernel(x, table):
    xt = x.T
    tt = table.T
    mesh = plsc.VectorSubcoreMesh(core_axis_name="c", subcore_axis_name="s")
    fn = pl.kernel(
        _gather_exp_body,
        out_type=jax.ShapeDtypeStruct((_S, _B), jnp.float32),
        mesh=mesh,
        scratch_types=[
            pltpu.VMEM((_STAGE_CHUNK,), jnp.float32),
            pltpu.VMEM((1, _SEG), jnp.int32),
            pltpu.VMEM((1, _SEG), jnp.int32),
            pltpu.VMEM((1, _SEG), jnp.float32),
            pltpu.VMEM((1, _SEG), jnp.float32),
            pltpu.VMEM_SHARED((_V,), jnp.float32),
            pltpu.SemaphoreType.DMA,
            pltpu.SemaphoreType.DMA,
            pltpu.SemaphoreType.DMA,
            pltpu.SemaphoreType.DMA,
            pltpu.SemaphoreType.DMA,
        ],
    )
    return fn(xt, tt).T


# speedup vs baseline: 281.3115x; 1.0000x over previous
import jax
import jax.numpy as jnp
from jax import lax
from jax.experimental import pallas as pl
from jax.experimental.pallas import tpu as pltpu
from jax.experimental.pallas import tpu_sc as plsc

_B = 16384
_S = 200
_N = _B * _S
_V = 1000000
_NC = 2
_NS = 16
_NW = _NC * _NS
_SEG = 2048
_SEG_PER_ROW = _B // _SEG
_NSEG = _S * _SEG_PER_ROW
_SEG_W = _NSEG // _NW
_PAIRS = _SEG_W // 2
_LANES = 16
_STAGE_CHUNK = 8192
_N_FULL_CHUNKS = _V // _STAGE_CHUNK
_STAGE_TAIL = _V - _N_FULL_CHUNKS * _STAGE_CHUNK
_N_STAGE_CHUNKS = _N_FULL_CHUNKS + 1
_STAGE_TRIPS = -(-_N_STAGE_CHUNKS // _NS)


def _gather_exp_body(x_hbm, table_hbm, out_hbm, stage_v, idx_a, idx_b,
                     rows_a, rows_b, etab_s, sem_g, sem_g2, sem_ia, sem_ib,
                     sem_oa, sem_ob):
    c = lax.axis_index("c")
    s = lax.axis_index("s")
    wid = s * _NC + c

    def _stage(off, size):
        pltpu.sync_copy(table_hbm.at[0, pl.ds(off, size)],
                        stage_v.at[pl.ds(0, size)])

        def exp_body(vi, cc):
            sl = pl.ds(pl.multiple_of(vi * _LANES, _LANES), _LANES)
            stage_v[sl] = jnp.exp(stage_v[sl])
            return cc

        lax.fori_loop(0, size // _LANES, exp_body, 0)
        pltpu.sync_copy(stage_v.at[pl.ds(0, size)],
                        etab_s.at[pl.ds(off, size)])

    def stage_body(k, carry):
        ci = k * _NS + s

        @pl.when(ci < _N_FULL_CHUNKS)
        def _():
            _stage(ci * _STAGE_CHUNK, _STAGE_CHUNK)

        @pl.when(ci == _N_FULL_CHUNKS)
        def _():
            _stage(_N_FULL_CHUNKS * _STAGE_CHUNK, _STAGE_TAIL)

        return carry

    lax.fori_loop(0, _STAGE_TRIPS, stage_body, 0)
    plsc.subcore_barrier()

    base = wid * _SEG_W

    def _x_slice(q):
        r = q // _SEG_PER_ROW
        b0 = (q % _SEG_PER_ROW) * _SEG
        return x_hbm.at[pl.ds(r, 1), pl.ds(b0, _SEG)]

    def _out_slice(q):
        r = q // _SEG_PER_ROW
        b0 = (q % _SEG_PER_ROW) * _SEG
        return out_hbm.at[pl.ds(r, 1), pl.ds(b0, _SEG)]

    def _run_seg(q, qnext, qprev, idx_c, rows_c, idx_n, rows_p, sem_ic,
                 sem_in, sem_oc, sem_op, j):
        pltpu.make_async_copy(_x_slice(q), idx_c, sem_ic).wait()

        @pl.when(qnext < base + _SEG_W)
        def _():
            pltpu.async_copy(_x_slice(qnext), idx_n, sem_in)

        @pl.when(j > 0)
        def _():
            pltpu.make_async_copy(rows_c, _out_slice(qprev), sem_oc).wait()

        _H = _SEG // 2
        h1 = pltpu.async_copy(etab_s.at[idx_c.at[0, pl.ds(0, _H)]],
                              rows_c.at[0, pl.ds(0, _H)], sem_g)
        h2 = pltpu.async_copy(etab_s.at[idx_c.at[0, pl.ds(_H, _H)]],
                              rows_c.at[0, pl.ds(_H, _H)], sem_g2)
        h1.wait()
        h2.wait()
        pltpu.async_copy(rows_c, _out_slice(q), sem_oc)

    pltpu.async_copy(_x_slice(base), idx_a, sem_ia)

    def pair_body(j, carry):
        q0 = base + 2 * j
        _run_seg(q0, q0 + 1, q0 - 2, idx_a, rows_a, idx_b, rows_b,
                 sem_ia, sem_ib, sem_oa, sem_ob, j)
        _run_seg(q0 + 1, q0 + 2, q0 - 1, idx_b, rows_b, idx_a, rows_a,
                 sem_ib, sem_ia, sem_ob, sem_oa, j)
        return carry

    lax.fori_loop(0, _PAIRS, pair_body, 0)
    pltpu.make_async_copy(rows_a, _out_slice(base + _SEG_W - 2),
                          sem_oa).wait()
    pltpu.make_async_copy(rows_b, _out_slice(base + _SEG_W - 1),
                          sem_ob).wait()


def kernel(x, table):
    xt = x.T
    tt = table.T
    mesh = plsc.VectorSubcoreMesh(core_axis_name="c", subcore_axis_name="s")
    fn = pl.kernel(
        _gather_exp_body,
        out_type=jax.ShapeDtypeStruct((_S, _B), jnp.float32),
        mesh=mesh,
        scratch_types=[
            pltpu.VMEM((_STAGE_CHUNK,), jnp.float32),
            pltpu.VMEM((1, _SEG), jnp.int32),
            pltpu.VMEM((1, _SEG), jnp.int32),
            pltpu.VMEM((1, _SEG), jnp.float32),
            pltpu.VMEM((1, _SEG), jnp.float32),
            pltpu.VMEM_SHARED((_V,), jnp.float32),
            pltpu.SemaphoreType.DMA,
            pltpu.SemaphoreType.DMA,
            pltpu.SemaphoreType.DMA,
            pltpu.SemaphoreType.DMA,
            pltpu.SemaphoreType.DMA,
            pltpu.SemaphoreType.DMA,
        ],
    )
    return fn(xt, tt).T
